# baseline (speedup 1.0000x reference)
import functools

import jax
import jax.numpy as jnp
from jax import lax
from jax.experimental import pallas as pl
from jax.experimental.pallas import tpu as pltpu
from jax.experimental.pallas import tpu_sc as plsc

N = 10000
E = 320000
ET = E + N
IN = 128
H = 8
C = 16
D = H * C
HID = 128

NC = 2
NS = 16
NW = NC * NS
CH = 64
NCHUNK = 162
NSLOT = 3
IW = 2 * CH
ET_PAD = NW * CH * NCHUNK
EPW = CH * NCHUNK
N_PAD = 10112
RPT = N_PAD // NS

BLK = 1000
GRID = N // BLK


def _s1_body(h_ref, W0_ref, W1_ref, As0_ref, Ad0_ref, As1_ref, Ad1_ref,
             xp0_ref, xp1_ref, als0_ref, ald0_ref, als1_ref, ald1_ref,
             m0_ref, m1_ref):
    i = pl.program_id(0)
    x = h_ref[...]
    for (W_ref, As_ref, Ad_ref, xp_ref, als_ref, ald_ref, m_ref) in (
            (W0_ref, As0_ref, Ad0_ref, xp0_ref, als0_ref, ald0_ref, m0_ref),
            (W1_ref, As1_ref, Ad1_ref, xp1_ref, als1_ref, ald1_ref, m1_ref)):
        xp = jnp.dot(x, W_ref[...], preferred_element_type=jnp.float32)
        xp_ref[...] = xp
        als = jnp.dot(xp, As_ref[...], preferred_element_type=jnp.float32)
        ald = jnp.dot(xp, Ad_ref[...], preferred_element_type=jnp.float32)
        als_ref[...] = als
        ald_ref[...] = ald
        bm = jnp.concatenate(
            [jnp.max(als, axis=0, keepdims=True),
             jnp.max(ald, axis=0, keepdims=True)], axis=0)

        @pl.when(i == 0)
        def _():
            m_ref[...] = bm

        @pl.when(i > 0)
        def _():
            m_ref[...] = jnp.maximum(m_ref[...], bm)


def _stage1(h, W0, W1, As0, Ad0, As1, Ad1):
    full = lambda i: (0, 0)
    row = lambda i: (i, 0)
    out = pl.pallas_call(
        _s1_body,
        grid=(GRID,),
        in_specs=[
            pl.BlockSpec((BLK, IN), row),
            pl.BlockSpec((IN, D), full), pl.BlockSpec((IN, D), full),
            pl.BlockSpec((D, H), full), pl.BlockSpec((D, H), full),
            pl.BlockSpec((D, H), full), pl.BlockSpec((D, H), full),
        ],
        out_specs=[
            pl.BlockSpec((BLK, D), row), pl.BlockSpec((BLK, D), row),
            pl.BlockSpec((BLK, H), row), pl.BlockSpec((BLK, H), row),
            pl.BlockSpec((BLK, H), row), pl.BlockSpec((BLK, H), row),
            pl.BlockSpec((2, H), full), pl.BlockSpec((2, H), full),
        ],
        out_shape=[
            jax.ShapeDtypeStruct((N, D), jnp.float32),
            jax.ShapeDtypeStruct((N, D), jnp.float32),
            jax.ShapeDtypeStruct((N, H), jnp.float32),
            jax.ShapeDtypeStruct((N, H), jnp.float32),
            jax.ShapeDtypeStruct((N, H), jnp.float32),
            jax.ShapeDtypeStruct((N, H), jnp.float32),
            jax.ShapeDtypeStruct((2, H), jnp.float32),
            jax.ShapeDtypeStruct((2, H), jnp.float32),
        ],
    )(h, W0, W1, As0, Ad0, As1, Ad1)
    return out


def _sc_body(xp0, als0, ald0, m0, idx0, xp1, als1, ald1, m1, idx1, zacc, zden,
             acc0_out, acc1_out, den0_out, den1_out,
             acc_sh, den_sh, buf, alsbuf, aldbuf, sbuf, idx_v, src_v, dst_v,
             mrep_v, semi, semg, semw):
    c = lax.axis_index("c")
    s = lax.axis_index("s")
    wid = c * NS + s
    r0 = s * RPT
    iota16 = lax.iota(jnp.int32, 16)

    for t in range(NSLOT):
        pltpu.sync_copy(zden.at[pl.ds(0, CH)], sbuf[t])

    for (xp_h, als_h, ald_h, m_h, idx_h, acc_out, den_out) in (
            (xp0, als0, ald0, m0, idx0, acc0_out, den0_out),
            (xp1, als1, ald1, m1, idx1, acc1_out, den1_out)):
        pltpu.sync_copy(zacc.at[pl.ds(r0, RPT)], acc_sh.at[pl.ds(r0, RPT)])
        pltpu.sync_copy(zden.at[pl.ds(r0, RPT)], den_sh.at[pl.ds(r0, RPT)])
        pltpu.sync_copy(m_h, mrep_v)
        plsc.subcore_barrier()
        mvecs = [mrep_v[hh, :] for hh in range(H)]

        def issue_idx(t, k):
            off = (k * NW + wid) * IW
            return pltpu.async_copy(idx_h.at[pl.ds(off, IW)], idx_v[t], semi[t])

        def wait_idx(t):
            pltpu.make_async_copy(
                idx_h.at[pl.ds(0, IW)], idx_v[t], semi[t]).wait()

        def prep(t, k):
            wait_idx(t)
            for i4 in range(CH // 16):
                src_v[t][pl.ds(i4 * 16, 16)] = idx_v[t][pl.ds(i4 * 16, 16)]
                dst_v[t][pl.ds(i4 * 16, 16)] = idx_v[t][pl.ds(CH + i4 * 16, 16)]
            pltpu.async_copy(xp_h.at[src_v[t]], buf[t], semg[t])
            pltpu.async_copy(als_h.at[src_v[t]], alsbuf[t], semg[t])
            pltpu.async_copy(ald_h.at[dst_v[t]], aldbuf[t], semg[t])
            issue_idx(t, k + NSLOT)

        def wait_gathers(t):
            pltpu.make_async_copy(xp_h.at[src_v[t]], buf[t], semg[t]).wait()
            pltpu.make_async_copy(als_h.at[src_v[t]], alsbuf[t], semg[t]).wait()
            pltpu.make_async_copy(ald_h.at[dst_v[t]], aldbuf[t], semg[t]).wait()

        def compute(t, k):
            base = (k * NW + wid) * CH

            def group_body(g, carry2):
                lidx = g * 16 + iota16
                valid = (base + lidx) < ET
                svecs = []
                for hh in range(H):
                    hvec = jnp.full((16,), hh, jnp.int32)
                    a_s = plsc.load_gather(alsbuf[t], [lidx, hvec])
                    a_d = plsc.load_gather(aldbuf[t], [lidx, hvec])
                    ev = a_s + a_d
                    ev = jnp.where(ev >= 0.0, ev, 0.2 * ev)
                    sv = jnp.exp(ev - mvecs[hh])
                    sv = jnp.where(valid, sv, 0.0)
                    svecs.append(sv)
                    plsc.store_scatter(sbuf[t], [lidx, hvec], sv)
                for e in range(16):
                    row = g * 16 + e
                    for hh in range(H):
                        sc = svecs[hh][e]
                        v = buf[t][row, pl.ds(hh * 16, 16)]
                        buf[t][row, pl.ds(hh * 16, 16)] = v * sc
                return carry2

            lax.fori_loop(0, CH // 16, group_body, 0)

        def issue_scatter(t):
            pltpu.async_copy(buf[t], acc_sh.at[dst_v[t]], semw[t], add=True)
            pltpu.async_copy(sbuf[t], den_sh.at[dst_v[t]], semw[t], add=True)

        def wait_scatter(t):
            pltpu.make_async_copy(buf[t], acc_sh.at[dst_v[t]], semw[t]).wait()
            pltpu.make_async_copy(sbuf[t], den_sh.at[dst_v[t]], semw[t]).wait()

        for t in range(NSLOT):
            issue_idx(t, t)
        for t in range(NSLOT):
            prep(t, t)

        def body(j, carry):
            k = j * NSLOT
            wait_gathers(0)
            compute(0, k)
            issue_scatter(0)
            wait_gathers(1)
            compute(1, k + 1)
            issue_scatter(1)
            wait_scatter(0)
            prep(0, k + NSLOT)
            wait_gathers(2)
            compute(2, k + 2)
            issue_scatter(2)
            wait_scatter(1)
            prep(1, k + NSLOT + 1)
            wait_scatter(2)
            prep(2, k + NSLOT + 2)
            return carry

        lax.fori_loop(0, NCHUNK // NSLOT, body, 0)
        for t in range(NSLOT):
            wait_gathers(t)
            wait_idx(t)
        plsc.subcore_barrier()
        pltpu.sync_copy(acc_sh.at[pl.ds(r0, RPT)], acc_out.at[c, pl.ds(r0, RPT)])
        pltpu.sync_copy(den_sh.at[pl.ds(r0, RPT)], den_out.at[c, pl.ds(r0, RPT)])
        plsc.subcore_barrier()


def _stage2(xp0, als0, ald0p, m0, idx0, xp1, als1, ald1p, m1, idx1):
    zacc = jnp.zeros((N_PAD, D), jnp.float32)
    zden = jnp.zeros((N_PAD, 16), jnp.float32)
    mesh = plsc.VectorSubcoreMesh(core_axis_name="c", subcore_axis_name="s")
    f32 = jnp.float32
    call = pl.kernel(
        _sc_body,
        out_type=[
            jax.ShapeDtypeStruct((NC, N_PAD, D), f32),
            jax.ShapeDtypeStruct((NC, N_PAD, D), f32),
            jax.ShapeDtypeStruct((NC, N_PAD, 16), f32),
            jax.ShapeDtypeStruct((NC, N_PAD, 16), f32),
        ],
        mesh=mesh,
        compiler_params=pltpu.CompilerParams(
            needs_layout_passes=False, use_tc_tiling_on_sc=False),
        scratch_types=[
            pltpu.VMEM_SHARED((N_PAD, D), f32),
            pltpu.VMEM_SHARED((N_PAD, 16), f32),
            [pltpu.VMEM((CH, D), f32)] * NSLOT,
            [pltpu.VMEM((CH, 16), f32)] * NSLOT,
            [pltpu.VMEM((CH, 16), f32)] * NSLOT,
            [pltpu.VMEM((CH, 16), f32)] * NSLOT,
            [pltpu.VMEM((IW,), jnp.int32)] * NSLOT,
            [pltpu.VMEM((CH,), jnp.int32)] * NSLOT,
            [pltpu.VMEM((CH,), jnp.int32)] * NSLOT,
            pltpu.VMEM((H, 16), f32),
            [pltpu.SemaphoreType.DMA] * NSLOT,
            [pltpu.SemaphoreType.DMA] * NSLOT,
            [pltpu.SemaphoreType.DMA] * NSLOT,
        ],
    )
    return call(xp0, als0, ald0p, m0, idx0, xp1, als1, ald1p, m1, idx1,
                zacc, zden)


def _s3_body(acc0_ref, acc1_ref, den0_ref, den1_ref, b0_ref, b1_ref, e16_ref,
             sw1_ref, sb1_ref, sw2_ref, out_ref):
    e16 = e16_ref[...]
    sw1 = sw1_ref[...]
    sb1 = sb1_ref[...]
    sw2 = sw2_ref[...]
    zs, ws = [], []
    for (acc_ref, den_ref, b_ref) in ((acc0_ref, den0_ref, b0_ref),
                                      (acc1_ref, den1_ref, b1_ref)):
        a = acc_ref[0] + acc_ref[1]
        dnm = den_ref[0] + den_ref[1]
        drep = jnp.dot(dnm, e16, preferred_element_type=jnp.float32)
        z = a / (drep + 1e-16) + b_ref[...]
        t = jnp.tanh(jnp.dot(z, sw1, preferred_element_type=jnp.float32) + sb1)
        w = jnp.sum(t * sw2, axis=1, keepdims=True)
        zs.append(z)
        ws.append(w)
    m = jnp.maximum(ws[0], ws[1])
    e0 = jnp.exp(ws[0] - m)
    e1 = jnp.exp(ws[1] - m)
    out_ref[...] = (e0 * zs[0] + e1 * zs[1]) / (e0 + e1)


def _stage3(acc0, acc1, den0, den1, b0, b1, e16, SW1, sb1, sw2r):
    full = lambda i: tuple(0 for _ in range(9))[:0]
    out = pl.pallas_call(
        _s3_body,
        grid=(GRID,),
        in_specs=[
            pl.BlockSpec((NC, BLK, D), lambda i: (0, i, 0)),
            pl.BlockSpec((NC, BLK, D), lambda i: (0, i, 0)),
            pl.BlockSpec((NC, BLK, 16), lambda i: (0, i, 0)),
            pl.BlockSpec((NC, BLK, 16), lambda i: (0, i, 0)),
            pl.BlockSpec((1, D), lambda i: (0, 0)),
            pl.BlockSpec((1, D), lambda i: (0, 0)),
            pl.BlockSpec((16, D), lambda i: (0, 0)),
            pl.BlockSpec((D, HID), lambda i: (0, 0)),
            pl.BlockSpec((1, HID), lambda i: (0, 0)),
            pl.BlockSpec((1, HID), lambda i: (0, 0)),
        ],
        out_specs=pl.BlockSpec((BLK, D), lambda i: (i, 0)),
        out_shape=jax.ShapeDtypeStruct((N, D), jnp.float32),
    )(acc0, acc1, den0, den1, b0, b1, e16, SW1, sb1, sw2r)
    return out


def kernel(h, gs, W0, A0s, A0d, B0, W1, A1s, A1d, B1, SW1, Sb1, SW2):
    eye = jnp.eye(H, dtype=jnp.float32)
    As0 = (A0s[:, :, None] * eye[:, None, :]).reshape(D, H)
    Ad0 = (A0d[:, :, None] * eye[:, None, :]).reshape(D, H)
    As1 = (A1s[:, :, None] * eye[:, None, :]).reshape(D, H)
    Ad1 = (A1d[:, :, None] * eye[:, None, :]).reshape(D, H)

    (xp0, xp1, als0, ald0, als1, ald1, m0, m1) = _stage1(
        h, W0, W1, As0, Ad0, As1, Ad1)

    mrep0 = jnp.broadcast_to((m0[0] + m0[1])[:, None], (H, 16))
    mrep1 = jnp.broadcast_to((m1[0] + m1[1])[:, None], (H, 16))
    als0p = jnp.pad(als0, ((0, 0), (0, 8)))
    als1p = jnp.pad(als1, ((0, 0), (0, 8)))
    ald0p = jnp.pad(ald0, ((0, 0), (0, 8)))
    ald1p = jnp.pad(ald1, ((0, 0), (0, 8)))

    loop = jnp.arange(N, dtype=jnp.int32)
    pad = jnp.zeros((ET_PAD - ET,), jnp.int32)
    nrow = ET_PAD // CH

    def packed_idx(src, dst):
        srcf = jnp.concatenate([src.astype(jnp.int32), loop, pad])
        dstf = jnp.concatenate([dst.astype(jnp.int32), loop, pad])
        rows = jnp.concatenate(
            [srcf.reshape(nrow, CH), dstf.reshape(nrow, CH)], axis=1)
        flat = rows.reshape(-1)
        return jnp.concatenate(
            [flat, jnp.zeros((3 * NW * IW,), jnp.int32)])

    idx0 = packed_idx(gs[0, 0], gs[0, 1])
    idx1 = packed_idx(gs[1, 0], gs[1, 1])

    acc0, acc1, den0, den1 = _stage2(
        xp0, als0p, ald0p, mrep0, idx0,
        xp1, als1p, ald1p, mrep1, idx1)

    e16 = jnp.repeat(jnp.eye(8, dtype=jnp.float32), 16, axis=1)
    e16 = jnp.concatenate([e16, jnp.zeros((8, D), jnp.float32)], axis=0)
    return _stage3(acc0, acc1, den0, den1,
                   B0.reshape(1, D), B1.reshape(1, D), e16,
                   SW1, Sb1.reshape(1, HID), SW2.reshape(1, HID))


# speedup vs baseline: 1.6042x; 1.6042x over previous
import functools

import jax
import jax.numpy as jnp
from jax import lax
from jax.experimental import pallas as pl
from jax.experimental.pallas import tpu as pltpu
from jax.experimental.pallas import tpu_sc as plsc

N = 10000
E = 320000
ET = E + N
IN = 128
H = 8
C = 16
D = H * C
HID = 128

NC = 2
NS = 16
NW = NC * NS
CH = 64
NCHUNK = 162
NSLOT = 3
IW = 2 * CH
ET_PAD = NW * CH * NCHUNK
EPW = CH * NCHUNK
N_PAD = 10112
RPT = N_PAD // NS

BLK = 1000
GRID = N // BLK


def _s1_body(h_ref, W0_ref, W1_ref, As0_ref, Ad0_ref, As1_ref, Ad1_ref,
             xp0_ref, xp1_ref, als0_ref, ald0_ref, als1_ref, ald1_ref,
             m0_ref, m1_ref):
    i = pl.program_id(0)
    x = h_ref[...]
    for (W_ref, As_ref, Ad_ref, xp_ref, als_ref, ald_ref, m_ref) in (
            (W0_ref, As0_ref, Ad0_ref, xp0_ref, als0_ref, ald0_ref, m0_ref),
            (W1_ref, As1_ref, Ad1_ref, xp1_ref, als1_ref, ald1_ref, m1_ref)):
        xp = jnp.dot(x, W_ref[...], preferred_element_type=jnp.float32)
        xp_ref[...] = xp
        als = jnp.dot(xp, As_ref[...], preferred_element_type=jnp.float32)
        ald = jnp.dot(xp, Ad_ref[...], preferred_element_type=jnp.float32)
        als_ref[...] = als
        ald_ref[...] = ald
        bm = jnp.concatenate(
            [jnp.max(als, axis=0, keepdims=True),
             jnp.max(ald, axis=0, keepdims=True)], axis=0)

        @pl.when(i == 0)
        def _():
            m_ref[...] = bm

        @pl.when(i > 0)
        def _():
            m_ref[...] = jnp.maximum(m_ref[...], bm)


def _stage1(h, W0, W1, As0, Ad0, As1, Ad1):
    full = lambda i: (0, 0)
    row = lambda i: (i, 0)
    out = pl.pallas_call(
        _s1_body,
        grid=(GRID,),
        in_specs=[
            pl.BlockSpec((BLK, IN), row),
            pl.BlockSpec((IN, D), full), pl.BlockSpec((IN, D), full),
            pl.BlockSpec((D, H), full), pl.BlockSpec((D, H), full),
            pl.BlockSpec((D, H), full), pl.BlockSpec((D, H), full),
        ],
        out_specs=[
            pl.BlockSpec((BLK, D), row), pl.BlockSpec((BLK, D), row),
            pl.BlockSpec((BLK, H), row), pl.BlockSpec((BLK, H), row),
            pl.BlockSpec((BLK, H), row), pl.BlockSpec((BLK, H), row),
            pl.BlockSpec((2, H), full), pl.BlockSpec((2, H), full),
        ],
        out_shape=[
            jax.ShapeDtypeStruct((N, D), jnp.float32),
            jax.ShapeDtypeStruct((N, D), jnp.float32),
            jax.ShapeDtypeStruct((N, H), jnp.float32),
            jax.ShapeDtypeStruct((N, H), jnp.float32),
            jax.ShapeDtypeStruct((N, H), jnp.float32),
            jax.ShapeDtypeStruct((N, H), jnp.float32),
            jax.ShapeDtypeStruct((2, H), jnp.float32),
            jax.ShapeDtypeStruct((2, H), jnp.float32),
        ],
    )(h, W0, W1, As0, Ad0, As1, Ad1)
    return out


def _sc_body(xp0, als0, ald0, m0, idx0, xp1, als1, ald1, m1, idx1, zacc, zden,
             acc0_out, acc1_out, den0_out, den1_out,
             acc_sh, den_sh, buf, alsbuf, aldbuf, sbuf, idx_v, src_v, dst_v,
             mrep_v, semi, semg, semw):
    c = lax.axis_index("c")
    s = lax.axis_index("s")
    wid = c * NS + s
    r0 = s * RPT
    iota16 = lax.iota(jnp.int32, 16)

    for t in range(NSLOT):
        pltpu.sync_copy(zden.at[pl.ds(0, CH)], sbuf[t])

    for (xp_h, als_h, ald_h, m_h, idx_h, acc_out, den_out) in (
            (xp0, als0, ald0, m0, idx0, acc0_out, den0_out),
            (xp1, als1, ald1, m1, idx1, acc1_out, den1_out)):
        pltpu.sync_copy(zacc.at[pl.ds(r0, RPT)], acc_sh.at[pl.ds(r0, RPT)])
        pltpu.sync_copy(zden.at[pl.ds(r0, RPT)], den_sh.at[pl.ds(r0, RPT)])
        pltpu.sync_copy(m_h, mrep_v)
        plsc.subcore_barrier()
        mvecs = [mrep_v[hh, :] for hh in range(H)]

        def issue_idx(t, k):
            off = (wid * NCHUNK + k) * IW
            return pltpu.async_copy(idx_h.at[pl.ds(off, IW)], idx_v[t], semi[t])

        def wait_idx(t):
            pltpu.make_async_copy(
                idx_h.at[pl.ds(0, IW)], idx_v[t], semi[t]).wait()

        def prep(t, k):
            wait_idx(t)
            for i4 in range(CH // 16):
                src_v[t][pl.ds(i4 * 16, 16)] = idx_v[t][pl.ds(i4 * 16, 16)]
                dst_v[t][pl.ds(i4 * 16, 16)] = idx_v[t][pl.ds(CH + i4 * 16, 16)]
            pltpu.async_copy(xp_h.at[src_v[t]], buf[t], semg[t])
            pltpu.async_copy(als_h.at[src_v[t]], alsbuf[t], semg[t])
            pltpu.async_copy(ald_h.at[dst_v[t]], aldbuf[t], semg[t])
            issue_idx(t, k + NSLOT)

        def wait_gathers(t):
            pltpu.make_async_copy(xp_h.at[src_v[t]], buf[t], semg[t]).wait()
            pltpu.make_async_copy(als_h.at[src_v[t]], alsbuf[t], semg[t]).wait()
            pltpu.make_async_copy(ald_h.at[dst_v[t]], aldbuf[t], semg[t]).wait()

        def compute(t, k):
            base = wid * EPW + k * CH

            def group_body(g, carry2):
                lidx = g * 16 + iota16
                valid = (base + lidx) < ET
                svecs = []
                for hh in range(H):
                    hvec = jnp.full((16,), hh, jnp.int32)
                    a_s = plsc.load_gather(alsbuf[t], [lidx, hvec])
                    a_d = plsc.load_gather(aldbuf[t], [lidx, hvec])
                    ev = a_s + a_d
                    ev = jnp.where(ev >= 0.0, ev, 0.2 * ev)
                    sv = jnp.exp(ev - mvecs[hh])
                    sv = jnp.where(valid, sv, 0.0)
                    svecs.append(sv)
                    plsc.store_scatter(sbuf[t], [lidx, hvec], sv)
                for e in range(16):
                    row = g * 16 + e
                    for hh in range(H):
                        sc = svecs[hh][e]
                        v = buf[t][row, pl.ds(hh * 16, 16)]
                        buf[t][row, pl.ds(hh * 16, 16)] = v * sc
                return carry2

            lax.fori_loop(0, CH // 16, group_body, 0)

        def issue_scatter(t):
            pltpu.async_copy(buf[t], acc_sh.at[dst_v[t]], semw[t], add=True)
            pltpu.async_copy(sbuf[t], den_sh.at[dst_v[t]], semw[t], add=True)

        def wait_scatter(t):
            pltpu.make_async_copy(buf[t], acc_sh.at[dst_v[t]], semw[t]).wait()
            pltpu.make_async_copy(sbuf[t], den_sh.at[dst_v[t]], semw[t]).wait()

        for t in range(NSLOT):
            issue_idx(t, t)
        for t in range(NSLOT):
            prep(t, t)

        def body(j, carry):
            k = j * NSLOT
            wait_gathers(0)
            compute(0, k)
            issue_scatter(0)
            wait_gathers(1)
            compute(1, k + 1)
            issue_scatter(1)
            wait_scatter(0)
            prep(0, k + NSLOT)
            wait_gathers(2)
            compute(2, k + 2)
            issue_scatter(2)
            wait_scatter(1)
            prep(1, k + NSLOT + 1)
            wait_scatter(2)
            prep(2, k + NSLOT + 2)
            return carry

        lax.fori_loop(0, NCHUNK // NSLOT, body, 0)
        for t in range(NSLOT):
            wait_gathers(t)
            wait_idx(t)
        plsc.subcore_barrier()
        pltpu.sync_copy(acc_sh.at[pl.ds(r0, RPT)], acc_out.at[c, pl.ds(r0, RPT)])
        pltpu.sync_copy(den_sh.at[pl.ds(r0, RPT)], den_out.at[c, pl.ds(r0, RPT)])
        plsc.subcore_barrier()


def _stage2(xp0, als0, ald0p, m0, idx0, xp1, als1, ald1p, m1, idx1):
    zacc = jnp.zeros((N_PAD, D), jnp.float32)
    zden = jnp.zeros((N_PAD, 16), jnp.float32)
    mesh = plsc.VectorSubcoreMesh(core_axis_name="c", subcore_axis_name="s")
    f32 = jnp.float32
    call = pl.kernel(
        _sc_body,
        out_type=[
            jax.ShapeDtypeStruct((NC, N_PAD, D), f32),
            jax.ShapeDtypeStruct((NC, N_PAD, D), f32),
            jax.ShapeDtypeStruct((NC, N_PAD, 16), f32),
            jax.ShapeDtypeStruct((NC, N_PAD, 16), f32),
        ],
        mesh=mesh,
        compiler_params=pltpu.CompilerParams(
            needs_layout_passes=False, use_tc_tiling_on_sc=False),
        scratch_types=[
            pltpu.VMEM_SHARED((N_PAD, D), f32),
            pltpu.VMEM_SHARED((N_PAD, 16), f32),
            [pltpu.VMEM((CH, D), f32)] * NSLOT,
            [pltpu.VMEM((CH, 16), f32)] * NSLOT,
            [pltpu.VMEM((CH, 16), f32)] * NSLOT,
            [pltpu.VMEM((CH, 16), f32)] * NSLOT,
            [pltpu.VMEM((IW,), jnp.int32)] * NSLOT,
            [pltpu.VMEM((CH,), jnp.int32)] * NSLOT,
            [pltpu.VMEM((CH,), jnp.int32)] * NSLOT,
            pltpu.VMEM((H, 16), f32),
            [pltpu.SemaphoreType.DMA] * NSLOT,
            [pltpu.SemaphoreType.DMA] * NSLOT,
            [pltpu.SemaphoreType.DMA] * NSLOT,
        ],
    )
    return call(xp0, als0, ald0p, m0, idx0, xp1, als1, ald1p, m1, idx1,
                zacc, zden)


def _s3_body(acc0_ref, acc1_ref, den0_ref, den1_ref, b0_ref, b1_ref, e16_ref,
             sw1_ref, sb1_ref, sw2_ref, out_ref):
    e16 = e16_ref[...]
    sw1 = sw1_ref[...]
    sb1 = sb1_ref[...]
    sw2 = sw2_ref[...]
    zs, ws = [], []
    for (acc_ref, den_ref, b_ref) in ((acc0_ref, den0_ref, b0_ref),
                                      (acc1_ref, den1_ref, b1_ref)):
        a = acc_ref[0] + acc_ref[1]
        dnm = den_ref[0] + den_ref[1]
        drep = jnp.dot(dnm, e16, preferred_element_type=jnp.float32)
        z = a / (drep + 1e-16) + b_ref[...]
        t = jnp.tanh(jnp.dot(z, sw1, preferred_element_type=jnp.float32) + sb1)
        w = jnp.sum(t * sw2, axis=1, keepdims=True)
        zs.append(z)
        ws.append(w)
    m = jnp.maximum(ws[0], ws[1])
    e0 = jnp.exp(ws[0] - m)
    e1 = jnp.exp(ws[1] - m)
    out_ref[...] = (e0 * zs[0] + e1 * zs[1]) / (e0 + e1)


def _stage3(acc0, acc1, den0, den1, b0, b1, e16, SW1, sb1, sw2r):
    full = lambda i: tuple(0 for _ in range(9))[:0]
    out = pl.pallas_call(
        _s3_body,
        grid=(GRID,),
        in_specs=[
            pl.BlockSpec((NC, BLK, D), lambda i: (0, i, 0)),
            pl.BlockSpec((NC, BLK, D), lambda i: (0, i, 0)),
            pl.BlockSpec((NC, BLK, 16), lambda i: (0, i, 0)),
            pl.BlockSpec((NC, BLK, 16), lambda i: (0, i, 0)),
            pl.BlockSpec((1, D), lambda i: (0, 0)),
            pl.BlockSpec((1, D), lambda i: (0, 0)),
            pl.BlockSpec((16, D), lambda i: (0, 0)),
            pl.BlockSpec((D, HID), lambda i: (0, 0)),
            pl.BlockSpec((1, HID), lambda i: (0, 0)),
            pl.BlockSpec((1, HID), lambda i: (0, 0)),
        ],
        out_specs=pl.BlockSpec((BLK, D), lambda i: (i, 0)),
        out_shape=jax.ShapeDtypeStruct((N, D), jnp.float32),
    )(acc0, acc1, den0, den1, b0, b1, e16, SW1, sb1, sw2r)
    return out


def kernel(h, gs, W0, A0s, A0d, B0, W1, A1s, A1d, B1, SW1, Sb1, SW2):
    eye = jnp.eye(H, dtype=jnp.float32)
    As0 = (A0s[:, :, None] * eye[:, None, :]).reshape(D, H)
    Ad0 = (A0d[:, :, None] * eye[:, None, :]).reshape(D, H)
    As1 = (A1s[:, :, None] * eye[:, None, :]).reshape(D, H)
    Ad1 = (A1d[:, :, None] * eye[:, None, :]).reshape(D, H)

    (xp0, xp1, als0, ald0, als1, ald1, m0, m1) = _stage1(
        h, W0, W1, As0, Ad0, As1, Ad1)

    mrep0 = jnp.broadcast_to((m0[0] + m0[1])[:, None], (H, 16))
    mrep1 = jnp.broadcast_to((m1[0] + m1[1])[:, None], (H, 16))
    als0p = jnp.pad(als0, ((0, 0), (0, 8)))
    als1p = jnp.pad(als1, ((0, 0), (0, 8)))
    ald0p = jnp.pad(ald0, ((0, 0), (0, 8)))
    ald1p = jnp.pad(ald1, ((0, 0), (0, 8)))

    loop = jnp.arange(N, dtype=jnp.int32)
    pad = jnp.zeros((ET_PAD - ET,), jnp.int32)
    nrow = ET_PAD // CH

    def packed_idx(src, dst):
        srcf = jnp.concatenate([src.astype(jnp.int32), loop, pad])
        dstf = jnp.concatenate([dst.astype(jnp.int32), loop, pad])
        rows = jnp.concatenate(
            [srcf.reshape(nrow, CH), dstf.reshape(nrow, CH)], axis=1)
        flat = rows.reshape(-1)
        return jnp.concatenate(
            [flat, jnp.zeros((3 * NW * IW,), jnp.int32)])

    idx0 = packed_idx(gs[0, 0], gs[0, 1])
    idx1 = packed_idx(gs[1, 0], gs[1, 1])

    acc0, acc1, den0, den1 = _stage2(
        xp0, als0p, ald0p, mrep0, idx0,
        xp1, als1p, ald1p, mrep1, idx1)

    e16 = jnp.repeat(jnp.eye(8, dtype=jnp.float32), 16, axis=1)
    e16 = jnp.concatenate([e16, jnp.zeros((8, D), jnp.float32)], axis=0)
    return _stage3(acc0, acc1, den0, den1,
                   B0.reshape(1, D), B1.reshape(1, D), e16,
                   SW1, Sb1.reshape(1, HID), SW2.reshape(1, HID))


# speedup vs baseline: 1.6538x; 1.0309x over previous
import functools

import jax
import jax.numpy as jnp
from jax import lax
from jax.experimental import pallas as pl
from jax.experimental.pallas import tpu as pltpu
from jax.experimental.pallas import tpu_sc as plsc

N = 10000
E = 320000
ET = E + N
IN = 128
H = 8
C = 16
D = H * C
HID = 128

NC = 2
NS = 16
NW = NC * NS
CH = 64
NCHUNK = 162
NSLOT = 3
IW = 2 * CH
ET_PAD = NW * CH * NCHUNK
EPW = CH * NCHUNK
N_PAD = 10112
RPT = N_PAD // NS

BLK = 1000
GRID = N // BLK


def _s1_body(h_ref, W0_ref, W1_ref, As0_ref, Ad0_ref, As1_ref, Ad1_ref,
             xp0_ref, xp1_ref, als0_ref, ald0_ref, als1_ref, ald1_ref,
             m0_ref, m1_ref):
    i = pl.program_id(0)
    x = h_ref[...]
    for (W_ref, As_ref, Ad_ref, xp_ref, als_ref, ald_ref, m_ref) in (
            (W0_ref, As0_ref, Ad0_ref, xp0_ref, als0_ref, ald0_ref, m0_ref),
            (W1_ref, As1_ref, Ad1_ref, xp1_ref, als1_ref, ald1_ref, m1_ref)):
        xp = jnp.dot(x, W_ref[...], preferred_element_type=jnp.float32)
        xp_ref[...] = xp
        als = jnp.dot(xp, As_ref[...], preferred_element_type=jnp.float32)
        ald = jnp.dot(xp, Ad_ref[...], preferred_element_type=jnp.float32)
        als_ref[...] = als
        ald_ref[...] = ald
        bm = jnp.concatenate(
            [jnp.max(als, axis=0, keepdims=True),
             jnp.max(ald, axis=0, keepdims=True)], axis=0)

        @pl.when(i == 0)
        def _():
            m_ref[...] = bm

        @pl.when(i > 0)
        def _():
            m_ref[...] = jnp.maximum(m_ref[...], bm)


def _stage1(h, W0, W1, As0, Ad0, As1, Ad1):
    full = lambda i: (0, 0)
    row = lambda i: (i, 0)
    out = pl.pallas_call(
        _s1_body,
        grid=(GRID,),
        in_specs=[
            pl.BlockSpec((BLK, IN), row),
            pl.BlockSpec((IN, D), full), pl.BlockSpec((IN, D), full),
            pl.BlockSpec((D, H), full), pl.BlockSpec((D, H), full),
            pl.BlockSpec((D, H), full), pl.BlockSpec((D, H), full),
        ],
        out_specs=[
            pl.BlockSpec((BLK, D), row), pl.BlockSpec((BLK, D), row),
            pl.BlockSpec((BLK, H), row), pl.BlockSpec((BLK, H), row),
            pl.BlockSpec((BLK, H), row), pl.BlockSpec((BLK, H), row),
            pl.BlockSpec((2, H), full), pl.BlockSpec((2, H), full),
        ],
        out_shape=[
            jax.ShapeDtypeStruct((N, D), jnp.float32),
            jax.ShapeDtypeStruct((N, D), jnp.float32),
            jax.ShapeDtypeStruct((N, H), jnp.float32),
            jax.ShapeDtypeStruct((N, H), jnp.float32),
            jax.ShapeDtypeStruct((N, H), jnp.float32),
            jax.ShapeDtypeStruct((N, H), jnp.float32),
            jax.ShapeDtypeStruct((2, H), jnp.float32),
            jax.ShapeDtypeStruct((2, H), jnp.float32),
        ],
    )(h, W0, W1, As0, Ad0, As1, Ad1)
    return out


def _sc_body(xp0, als0, ald0, m0, idx0, xp1, als1, ald1, m1, idx1, zacc, zden,
             acc0_out, acc1_out, den0_out, den1_out,
             acc_sh, den_sh, buf, alsbuf, aldbuf, sbuf, idx_v, src_v, dst_v,
             mrep_v, semi, semg, semw):
    c = lax.axis_index("c")
    s = lax.axis_index("s")
    wid = c * NS + s
    r0 = s * RPT
    iota16 = lax.iota(jnp.int32, 16)

    for t in range(NSLOT):
        pltpu.sync_copy(zden.at[pl.ds(0, CH)], sbuf[t])

    for (xp_h, als_h, ald_h, m_h, idx_h, acc_out, den_out) in (
            (xp0, als0, ald0, m0, idx0, acc0_out, den0_out),
            (xp1, als1, ald1, m1, idx1, acc1_out, den1_out)):
        pltpu.sync_copy(zacc.at[pl.ds(r0, RPT)], acc_sh.at[pl.ds(r0, RPT)])
        pltpu.sync_copy(zden.at[pl.ds(r0, RPT)], den_sh.at[pl.ds(r0, RPT)])
        pltpu.sync_copy(m_h, mrep_v)
        plsc.subcore_barrier()
        mvecs = [mrep_v[hh, :] for hh in range(H)]

        def issue_idx(t, k):
            off = (wid * NCHUNK + k) * IW
            return pltpu.async_copy(idx_h.at[pl.ds(off, IW)], idx_v[t], semi[t])

        def wait_idx(t):
            pltpu.make_async_copy(
                idx_h.at[pl.ds(0, IW)], idx_v[t], semi[t]).wait()

        def prep(t, k):
            wait_idx(t)
            for i4 in range(CH // 16):
                src_v[t][pl.ds(i4 * 16, 16)] = idx_v[t][pl.ds(i4 * 16, 16)]
                dst_v[t][pl.ds(i4 * 16, 16)] = idx_v[t][pl.ds(CH + i4 * 16, 16)]
            pltpu.async_copy(xp_h.at[src_v[t]], buf[t], semg[t])
            pltpu.async_copy(als_h.at[src_v[t]], alsbuf[t], semg[t])
            pltpu.async_copy(ald_h.at[dst_v[t]], aldbuf[t], semg[t])
            issue_idx(t, k + NSLOT)

        def wait_gathers(t):
            pltpu.make_async_copy(xp_h.at[src_v[t]], buf[t], semg[t]).wait()
            pltpu.make_async_copy(als_h.at[src_v[t]], alsbuf[t], semg[t]).wait()
            pltpu.make_async_copy(ald_h.at[dst_v[t]], aldbuf[t], semg[t]).wait()

        def compute(t, k):
            base = wid * EPW + k * CH

            @plsc.parallel_loop(0, CH // 16, 1, unroll=2)
            def group_body(g):
                lidx = g * 16 + iota16
                valid = (base + lidx) < ET
                svecs = []
                for hh in range(H):
                    hvec = jnp.full((16,), hh, jnp.int32)
                    a_s = plsc.load_gather(alsbuf[t], [lidx, hvec])
                    a_d = plsc.load_gather(aldbuf[t], [lidx, hvec])
                    ev = a_s + a_d
                    ev = jnp.where(ev >= 0.0, ev, 0.2 * ev)
                    sv = jnp.exp(ev - mvecs[hh])
                    sv = jnp.where(valid, sv, 0.0)
                    svecs.append(sv)
                    plsc.store_scatter(sbuf[t], [lidx, hvec], sv)
                for e in range(16):
                    row = g * 16 + e
                    for hh in range(H):
                        sc = svecs[hh][e]
                        v = buf[t][row, pl.ds(hh * 16, 16)]
                        buf[t][row, pl.ds(hh * 16, 16)] = v * sc

        def issue_scatter(t):
            pltpu.async_copy(buf[t], acc_sh.at[dst_v[t]], semw[t], add=True)
            pltpu.async_copy(sbuf[t], den_sh.at[dst_v[t]], semw[t], add=True)

        def wait_scatter(t):
            pltpu.make_async_copy(buf[t], acc_sh.at[dst_v[t]], semw[t]).wait()
            pltpu.make_async_copy(sbuf[t], den_sh.at[dst_v[t]], semw[t]).wait()

        for t in range(NSLOT):
            issue_idx(t, t)
        for t in range(NSLOT):
            prep(t, t)

        def body(j, carry):
            k = j * NSLOT
            wait_gathers(0)
            compute(0, k)
            issue_scatter(0)
            wait_gathers(1)
            compute(1, k + 1)
            issue_scatter(1)
            wait_scatter(0)
            prep(0, k + NSLOT)
            wait_gathers(2)
            compute(2, k + 2)
            issue_scatter(2)
            wait_scatter(1)
            prep(1, k + NSLOT + 1)
            wait_scatter(2)
            prep(2, k + NSLOT + 2)
            return carry

        lax.fori_loop(0, NCHUNK // NSLOT, body, 0)
        for t in range(NSLOT):
            wait_gathers(t)
            wait_idx(t)
        plsc.subcore_barrier()
        pltpu.sync_copy(acc_sh.at[pl.ds(r0, RPT)], acc_out.at[c, pl.ds(r0, RPT)])
        pltpu.sync_copy(den_sh.at[pl.ds(r0, RPT)], den_out.at[c, pl.ds(r0, RPT)])
        plsc.subcore_barrier()


def _stage2(xp0, als0, ald0p, m0, idx0, xp1, als1, ald1p, m1, idx1):
    zacc = jnp.zeros((N_PAD, D), jnp.float32)
    zden = jnp.zeros((N_PAD, 16), jnp.float32)
    mesh = plsc.VectorSubcoreMesh(core_axis_name="c", subcore_axis_name="s")
    f32 = jnp.float32
    call = pl.kernel(
        _sc_body,
        out_type=[
            jax.ShapeDtypeStruct((NC, N_PAD, D), f32),
            jax.ShapeDtypeStruct((NC, N_PAD, D), f32),
            jax.ShapeDtypeStruct((NC, N_PAD, 16), f32),
            jax.ShapeDtypeStruct((NC, N_PAD, 16), f32),
        ],
        mesh=mesh,
        compiler_params=pltpu.CompilerParams(
            needs_layout_passes=False, use_tc_tiling_on_sc=False),
        scratch_types=[
            pltpu.VMEM_SHARED((N_PAD, D), f32),
            pltpu.VMEM_SHARED((N_PAD, 16), f32),
            [pltpu.VMEM((CH, D), f32)] * NSLOT,
            [pltpu.VMEM((CH, 16), f32)] * NSLOT,
            [pltpu.VMEM((CH, 16), f32)] * NSLOT,
            [pltpu.VMEM((CH, 16), f32)] * NSLOT,
            [pltpu.VMEM((IW,), jnp.int32)] * NSLOT,
            [pltpu.VMEM((CH,), jnp.int32)] * NSLOT,
            [pltpu.VMEM((CH,), jnp.int32)] * NSLOT,
            pltpu.VMEM((H, 16), f32),
            [pltpu.SemaphoreType.DMA] * NSLOT,
            [pltpu.SemaphoreType.DMA] * NSLOT,
            [pltpu.SemaphoreType.DMA] * NSLOT,
        ],
    )
    return call(xp0, als0, ald0p, m0, idx0, xp1, als1, ald1p, m1, idx1,
                zacc, zden)


def _s3_body(acc0_ref, acc1_ref, den0_ref, den1_ref, b0_ref, b1_ref, e16_ref,
             sw1_ref, sb1_ref, sw2_ref, out_ref):
    e16 = e16_ref[...]
    sw1 = sw1_ref[...]
    sb1 = sb1_ref[...]
    sw2 = sw2_ref[...]
    zs, ws = [], []
    for (acc_ref, den_ref, b_ref) in ((acc0_ref, den0_ref, b0_ref),
                                      (acc1_ref, den1_ref, b1_ref)):
        a = acc_ref[0] + acc_ref[1]
        dnm = den_ref[0] + den_ref[1]
        drep = jnp.dot(dnm, e16, preferred_element_type=jnp.float32)
        z = a / (drep + 1e-16) + b_ref[...]
        t = jnp.tanh(jnp.dot(z, sw1, preferred_element_type=jnp.float32) + sb1)
        w = jnp.sum(t * sw2, axis=1, keepdims=True)
        zs.append(z)
        ws.append(w)
    m = jnp.maximum(ws[0], ws[1])
    e0 = jnp.exp(ws[0] - m)
    e1 = jnp.exp(ws[1] - m)
    out_ref[...] = (e0 * zs[0] + e1 * zs[1]) / (e0 + e1)


def _stage3(acc0, acc1, den0, den1, b0, b1, e16, SW1, sb1, sw2r):
    full = lambda i: tuple(0 for _ in range(9))[:0]
    out = pl.pallas_call(
        _s3_body,
        grid=(GRID,),
        in_specs=[
            pl.BlockSpec((NC, BLK, D), lambda i: (0, i, 0)),
            pl.BlockSpec((NC, BLK, D), lambda i: (0, i, 0)),
            pl.BlockSpec((NC, BLK, 16), lambda i: (0, i, 0)),
            pl.BlockSpec((NC, BLK, 16), lambda i: (0, i, 0)),
            pl.BlockSpec((1, D), lambda i: (0, 0)),
            pl.BlockSpec((1, D), lambda i: (0, 0)),
            pl.BlockSpec((16, D), lambda i: (0, 0)),
            pl.BlockSpec((D, HID), lambda i: (0, 0)),
            pl.BlockSpec((1, HID), lambda i: (0, 0)),
            pl.BlockSpec((1, HID), lambda i: (0, 0)),
        ],
        out_specs=pl.BlockSpec((BLK, D), lambda i: (i, 0)),
        out_shape=jax.ShapeDtypeStruct((N, D), jnp.float32),
    )(acc0, acc1, den0, den1, b0, b1, e16, SW1, sb1, sw2r)
    return out


def kernel(h, gs, W0, A0s, A0d, B0, W1, A1s, A1d, B1, SW1, Sb1, SW2):
    eye = jnp.eye(H, dtype=jnp.float32)
    As0 = (A0s[:, :, None] * eye[:, None, :]).reshape(D, H)
    Ad0 = (A0d[:, :, None] * eye[:, None, :]).reshape(D, H)
    As1 = (A1s[:, :, None] * eye[:, None, :]).reshape(D, H)
    Ad1 = (A1d[:, :, None] * eye[:, None, :]).reshape(D, H)

    (xp0, xp1, als0, ald0, als1, ald1, m0, m1) = _stage1(
        h, W0, W1, As0, Ad0, As1, Ad1)

    mrep0 = jnp.broadcast_to((m0[0] + m0[1])[:, None], (H, 16))
    mrep1 = jnp.broadcast_to((m1[0] + m1[1])[:, None], (H, 16))
    als0p = jnp.pad(als0, ((0, 0), (0, 8)))
    als1p = jnp.pad(als1, ((0, 0), (0, 8)))
    ald0p = jnp.pad(ald0, ((0, 0), (0, 8)))
    ald1p = jnp.pad(ald1, ((0, 0), (0, 8)))

    loop = jnp.arange(N, dtype=jnp.int32)
    pad = jnp.zeros((ET_PAD - ET,), jnp.int32)
    nrow = ET_PAD // CH

    def packed_idx(src, dst):
        srcf = jnp.concatenate([src.astype(jnp.int32), loop, pad])
        dstf = jnp.concatenate([dst.astype(jnp.int32), loop, pad])
        rows = jnp.concatenate(
            [srcf.reshape(nrow, CH), dstf.reshape(nrow, CH)], axis=1)
        flat = rows.reshape(-1)
        return jnp.concatenate(
            [flat, jnp.zeros((3 * NW * IW,), jnp.int32)])

    idx0 = packed_idx(gs[0, 0], gs[0, 1])
    idx1 = packed_idx(gs[1, 0], gs[1, 1])

    acc0, acc1, den0, den1 = _stage2(
        xp0, als0p, ald0p, mrep0, idx0,
        xp1, als1p, ald1p, mrep1, idx1)

    e16 = jnp.repeat(jnp.eye(8, dtype=jnp.float32), 16, axis=1)
    e16 = jnp.concatenate([e16, jnp.zeros((8, D), jnp.float32)], axis=0)
    return _stage3(acc0, acc1, den0, den1,
                   B0.reshape(1, D), B1.reshape(1, D), e16,
                   SW1, Sb1.reshape(1, HID), SW2.reshape(1, HID))
